# baseline (speedup 1.0000x reference)
import functools

import jax
import jax.numpy as jnp
from jax.experimental import pallas as pl
from jax.experimental.pallas import tpu as pltpu

_N = 8192
_BM = 512


def _chamfer_body(a_ref, bt_ref, out_ref, d1_acc, s0_acc):
    i = pl.program_id(0)
    ni = pl.num_programs(0)

    a = a_ref[...]
    bt = bt_ref[...]
    n0 = 0.25 * jnp.sum(a * a, axis=1, keepdims=True)
    n1 = 0.25 * jnp.sum(bt * bt, axis=0, keepdims=True)
    prod = jnp.dot(a, bt, preferred_element_type=jnp.float32)

    row_min = jnp.min(prod + n1, axis=1) + n0[:, 0]
    col_min = jnp.min(prod + n0, axis=0, keepdims=True)

    @pl.when(i == 0)
    def _init():
        d1_acc[...] = col_min
        s0_acc[0, 0] = 0.0

    @pl.when(i != 0)
    def _accum():
        d1_acc[...] = jnp.minimum(d1_acc[...], col_min)

    s0_acc[0, 0] += jnp.sum(jnp.maximum(row_min, 0.0))

    @pl.when(i == ni - 1)
    def _finish():
        d1_sum = jnp.sum(jnp.maximum(d1_acc[...] + n1, 0.0))
        loss = (s0_acc[0, 0] + d1_sum) / float(_N)
        out_ref[...] = jnp.broadcast_to(loss, (1, 1))


def _chamfer(pc0, pc1t):
    ni = _N // _BM
    out = pl.pallas_call(
        _chamfer_body,
        grid=(ni,),
        in_specs=[
            pl.BlockSpec((_BM, 3), lambda i: (i, 0)),
            pl.BlockSpec((3, _N), lambda i: (0, 0)),
        ],
        out_specs=pl.BlockSpec((1, 1), lambda i: (0, 0)),
        out_shape=jax.ShapeDtypeStruct((1, 1), jnp.float32),
        scratch_shapes=[
            pltpu.VMEM((1, _N), jnp.float32),
            pltpu.SMEM((1, 1), jnp.float32),
        ],
    )(pc0, pc1t)
    return out[0, 0]


@jax.jit
def kernel(input0, input1):
    return _chamfer(-2.0 * input0, input1.T)


# speedup vs baseline: 1.0006x; 1.0006x over previous
import functools

import jax
import jax.numpy as jnp
from jax.experimental import pallas as pl
from jax.experimental.pallas import tpu as pltpu

_N = 8192
_BM = 512


def _chamfer_body(a_ref, bt_ref, out_ref, d1_acc, s0_acc):
    i = pl.program_id(0)
    ni = pl.num_programs(0)

    a = a_ref[...]
    bt = bt_ref[...]
    n0 = 0.25 * jnp.sum(a * a, axis=1, keepdims=True)
    n1 = jnp.sum(bt * bt, axis=0, keepdims=True)
    prod = jnp.dot(a, bt, preferred_element_type=jnp.float32)

    row_min = jnp.min(prod + n1, axis=1) + n0[:, 0]
    col_min = jnp.min(prod + n0, axis=0, keepdims=True)

    @pl.when(i == 0)
    def _init():
        d1_acc[...] = col_min
        s0_acc[0, 0] = 0.0

    @pl.when(i != 0)
    def _accum():
        d1_acc[...] = jnp.minimum(d1_acc[...], col_min)

    s0_acc[0, 0] += jnp.sum(jnp.maximum(row_min, 0.0))

    @pl.when(i == ni - 1)
    def _finish():
        d1_sum = jnp.sum(jnp.maximum(d1_acc[...] + n1, 0.0))
        loss = (s0_acc[0, 0] + d1_sum) / float(_N)
        out_ref[...] = jnp.broadcast_to(loss, (1, 1))


def _chamfer(pc0, pc1t):
    ni = _N // _BM
    out = pl.pallas_call(
        _chamfer_body,
        grid=(ni,),
        in_specs=[
            pl.BlockSpec((_BM, 3), lambda i: (i, 0)),
            pl.BlockSpec((3, _N), lambda i: (0, 0)),
        ],
        out_specs=pl.BlockSpec((1, 1), lambda i: (0, 0)),
        out_shape=jax.ShapeDtypeStruct((1, 1), jnp.float32),
        scratch_shapes=[
            pltpu.VMEM((1, _N), jnp.float32),
            pltpu.SMEM((1, 1), jnp.float32),
        ],
    )(pc0, pc1t)
    return out[0, 0]


@jax.jit
def kernel(input0, input1):
    return _chamfer(-2.0 * input0, input1.T)


# speedup vs baseline: 1.0151x; 1.0145x over previous
import functools

import jax
import jax.numpy as jnp
from jax.experimental import pallas as pl
from jax.experimental.pallas import tpu as pltpu

_N = 8192
_BM = 512


def _chamfer_body(a_ref, bt_ref, out_ref, d1_acc, s0_acc):
    i = pl.program_id(0)
    ni = pl.num_programs(0)

    a = a_ref[...]
    bt = bt_ref[...]
    n0 = jnp.sum(a * a, axis=1, keepdims=True)
    n1 = jnp.sum(bt * bt, axis=0, keepdims=True)
    prod = jnp.dot(-2.0 * a, bt, preferred_element_type=jnp.float32)

    row_min = jnp.min(prod + n1, axis=1) + n0[:, 0]
    col_min = jnp.min(prod + n0, axis=0, keepdims=True)

    @pl.when(i == 0)
    def _init():
        d1_acc[...] = col_min
        s0_acc[0, 0] = 0.0

    @pl.when(i != 0)
    def _accum():
        d1_acc[...] = jnp.minimum(d1_acc[...], col_min)

    s0_acc[0, 0] += jnp.sum(jnp.maximum(row_min, 0.0))

    @pl.when(i == ni - 1)
    def _finish():
        d1_sum = jnp.sum(jnp.maximum(d1_acc[...] + n1, 0.0))
        loss = (s0_acc[0, 0] + d1_sum) / float(_N)
        out_ref[...] = jnp.broadcast_to(loss, (1, 1))


def _chamfer(pc0, pc1t):
    ni = _N // _BM
    out = pl.pallas_call(
        _chamfer_body,
        grid=(ni,),
        in_specs=[
            pl.BlockSpec((_BM, 3), lambda i: (i, 0)),
            pl.BlockSpec((3, _N), lambda i: (0, 0)),
        ],
        out_specs=pl.BlockSpec((1, 1), lambda i: (0, 0)),
        out_shape=jax.ShapeDtypeStruct((1, 1), jnp.float32),
        scratch_shapes=[
            pltpu.VMEM((1, _N), jnp.float32),
            pltpu.SMEM((1, 1), jnp.float32),
        ],
    )(pc0, pc1t)
    return out[0, 0]


@jax.jit
def kernel(input0, input1):
    return _chamfer(input0, input1.T)


# speedup vs baseline: 1.0876x; 1.0714x over previous
import functools

import jax
import jax.numpy as jnp
from jax.experimental import pallas as pl
from jax.experimental.pallas import tpu as pltpu

_N = 8192
_BM = 1024


def _chamfer_body(a_ref, bt_ref, out_ref, d1_acc, s0_acc):
    i = pl.program_id(0)
    ni = pl.num_programs(0)

    a = a_ref[...]
    bt = bt_ref[...]
    n0 = jnp.sum(a * a, axis=1, keepdims=True)
    n1 = jnp.sum(bt * bt, axis=0, keepdims=True)
    prod = jnp.dot(-2.0 * a, bt, preferred_element_type=jnp.float32)

    row_min = jnp.min(prod + n1, axis=1) + n0[:, 0]
    col_min = jnp.min(prod + n0, axis=0, keepdims=True)

    @pl.when(i == 0)
    def _init():
        d1_acc[...] = col_min
        s0_acc[0, 0] = 0.0

    @pl.when(i != 0)
    def _accum():
        d1_acc[...] = jnp.minimum(d1_acc[...], col_min)

    s0_acc[0, 0] += jnp.sum(jnp.maximum(row_min, 0.0))

    @pl.when(i == ni - 1)
    def _finish():
        d1_sum = jnp.sum(jnp.maximum(d1_acc[...] + n1, 0.0))
        loss = (s0_acc[0, 0] + d1_sum) / float(_N)
        out_ref[...] = jnp.broadcast_to(loss, (1, 1))


def _chamfer(pc0, pc1t):
    ni = _N // _BM
    out = pl.pallas_call(
        _chamfer_body,
        grid=(ni,),
        in_specs=[
            pl.BlockSpec((_BM, 3), lambda i: (i, 0)),
            pl.BlockSpec((3, _N), lambda i: (0, 0)),
        ],
        out_specs=pl.BlockSpec((1, 1), lambda i: (0, 0)),
        out_shape=jax.ShapeDtypeStruct((1, 1), jnp.float32),
        scratch_shapes=[
            pltpu.VMEM((1, _N), jnp.float32),
            pltpu.SMEM((1, 1), jnp.float32),
        ],
    )(pc0, pc1t)
    return out[0, 0]


@jax.jit
def kernel(input0, input1):
    return _chamfer(input0, input1.T)


# speedup vs baseline: 1.1085x; 1.0192x over previous
import functools

import jax
import jax.numpy as jnp
from jax.experimental import pallas as pl
from jax.experimental.pallas import tpu as pltpu

_N = 8192
_BM = 2048


def _chamfer_body(a_ref, bt_ref, out_ref, d1_acc, s0_acc):
    i = pl.program_id(0)
    ni = pl.num_programs(0)

    a = a_ref[...]
    bt = bt_ref[...]
    n0 = jnp.sum(a * a, axis=1, keepdims=True)
    n1 = jnp.sum(bt * bt, axis=0, keepdims=True)
    prod = jnp.dot(-2.0 * a, bt, preferred_element_type=jnp.float32)

    row_min = jnp.min(prod + n1, axis=1) + n0[:, 0]
    col_min = jnp.min(prod + n0, axis=0, keepdims=True)

    @pl.when(i == 0)
    def _init():
        d1_acc[...] = col_min
        s0_acc[0, 0] = 0.0

    @pl.when(i != 0)
    def _accum():
        d1_acc[...] = jnp.minimum(d1_acc[...], col_min)

    s0_acc[0, 0] += jnp.sum(jnp.maximum(row_min, 0.0))

    @pl.when(i == ni - 1)
    def _finish():
        d1_sum = jnp.sum(jnp.maximum(d1_acc[...] + n1, 0.0))
        loss = (s0_acc[0, 0] + d1_sum) / float(_N)
        out_ref[...] = jnp.broadcast_to(loss, (1, 1))


def _chamfer(pc0, pc1t):
    ni = _N // _BM
    out = pl.pallas_call(
        _chamfer_body,
        grid=(ni,),
        in_specs=[
            pl.BlockSpec((_BM, 3), lambda i: (i, 0)),
            pl.BlockSpec((3, _N), lambda i: (0, 0)),
        ],
        out_specs=pl.BlockSpec((1, 1), lambda i: (0, 0)),
        out_shape=jax.ShapeDtypeStruct((1, 1), jnp.float32),
        scratch_shapes=[
            pltpu.VMEM((1, _N), jnp.float32),
            pltpu.SMEM((1, 1), jnp.float32),
        ],
    )(pc0, pc1t)
    return out[0, 0]


@jax.jit
def kernel(input0, input1):
    return _chamfer(input0, input1.T)


# speedup vs baseline: 1.1361x; 1.0248x over previous
import functools

import jax
import jax.numpy as jnp
from jax.experimental import pallas as pl
from jax.experimental.pallas import tpu as pltpu

_N = 8192
_BM = 4096


def _chamfer_body(a_ref, bt_ref, out_ref, d1_acc, s0_acc):
    i = pl.program_id(0)
    ni = pl.num_programs(0)

    a = a_ref[...]
    bt = bt_ref[...]
    n0 = jnp.sum(a * a, axis=1, keepdims=True)
    n1 = jnp.sum(bt * bt, axis=0, keepdims=True)
    prod = jnp.dot(-2.0 * a, bt, preferred_element_type=jnp.float32)

    row_min = jnp.min(prod + n1, axis=1) + n0[:, 0]
    col_min = jnp.min(prod + n0, axis=0, keepdims=True)

    @pl.when(i == 0)
    def _init():
        d1_acc[...] = col_min
        s0_acc[0, 0] = 0.0

    @pl.when(i != 0)
    def _accum():
        d1_acc[...] = jnp.minimum(d1_acc[...], col_min)

    s0_acc[0, 0] += jnp.sum(jnp.maximum(row_min, 0.0))

    @pl.when(i == ni - 1)
    def _finish():
        d1_sum = jnp.sum(jnp.maximum(d1_acc[...] + n1, 0.0))
        loss = (s0_acc[0, 0] + d1_sum) / float(_N)
        out_ref[...] = jnp.broadcast_to(loss, (1, 1))


def _chamfer(pc0, pc1t):
    ni = _N // _BM
    out = pl.pallas_call(
        _chamfer_body,
        grid=(ni,),
        in_specs=[
            pl.BlockSpec((_BM, 3), lambda i: (i, 0)),
            pl.BlockSpec((3, _N), lambda i: (0, 0)),
        ],
        out_specs=pl.BlockSpec((1, 1), lambda i: (0, 0)),
        out_shape=jax.ShapeDtypeStruct((1, 1), jnp.float32),
        scratch_shapes=[
            pltpu.VMEM((1, _N), jnp.float32),
            pltpu.SMEM((1, 1), jnp.float32),
        ],
    )(pc0, pc1t)
    return out[0, 0]


@jax.jit
def kernel(input0, input1):
    return _chamfer(input0, input1.T)
